# baseline (speedup 1.0000x reference)
import functools

import jax
import jax.numpy as jnp
from jax import lax
from jax.experimental import pallas as pl
from jax.experimental.pallas import tpu as pltpu
from jax.experimental.pallas import tpu_sc as plsc

_SHORT = 65536
_MED = 16384
_LONG = 4096
_DIM = 512
_TOTAL = _SHORT + _MED + _LONG
_B = 8192

_L = 16
_NC = 2
_NS = 16
_NW = _NC * _NS

_SLICE = _SHORT // _NW
_MSLICE = _MED // _NW
_LSLICE = _LONG // _NW
_NCHUNK = _B // _L
_WCHUNK = _SLICE // _L

_LOG_B = 13
_LOG_SLICE = 11


def _sc_body(upd_hbm, idx_hbm, short_hbm, med_hbm, long_hbm, out_hbm,
             idx_v, w_v, cl_v, shift_v, row_buf, sem_cp, sem_idx, sem_io):
    cid = lax.axis_index("c")
    sid = lax.axis_index("s")
    wid = sid * _NC + cid

    short_cp = pltpu.make_async_copy(
        short_hbm.at[pl.ds(wid * _SLICE, _SLICE)],
        out_hbm.at[pl.ds(wid * _SLICE, _SLICE)], sem_cp)
    med_cp = pltpu.make_async_copy(
        med_hbm.at[pl.ds(wid * _MSLICE, _MSLICE)],
        out_hbm.at[pl.ds(_SHORT + wid * _MSLICE, _MSLICE)], sem_cp)
    long_cp = pltpu.make_async_copy(
        long_hbm.at[pl.ds(wid * _LSLICE, _LSLICE)],
        out_hbm.at[pl.ds(_SHORT + _MED + wid * _LSLICE, _LSLICE)], sem_cp)
    short_cp.start()
    med_cp.start()
    long_cp.start()

    idx_cp = pltpu.make_async_copy(idx_hbm, idx_v, sem_idx)
    idx_cp.start()
    idx_cp.wait()

    iota = lax.iota(jnp.int32, _L)
    minus1 = jnp.full((_L,), -1, jnp.int32)

    def init_body(v, carry):
        w_v[pl.ds(v * _L, _L)] = minus1
        return carry

    lax.fori_loop(0, _WCHUNK, init_body, 0, unroll=False)

    def scan_body(c, carry):
        iv = idx_v[pl.ds(c * _L, _L)]
        shift_v[...] = iv
        drop = iota < 0
        for sh in range(1, _L):
            nb = plsc.load_gather(shift_v, [jnp.minimum(iota + sh, _L - 1)])
            drop = drop | ((iv == nb) & (iota + sh < _L))
        mine = lax.shift_right_logical(iv, _LOG_SLICE) == wid
        keep = mine & jnp.logical_not(drop)
        addr = jnp.where(keep, iv & (_SLICE - 1), _SLICE + iota)
        plsc.store_scatter(w_v, [addr], iota + c * _L)
        return carry

    lax.fori_loop(0, _NCHUNK, scan_body, 0, unroll=False)

    def compact_body(v, base):
        wv = w_v[pl.ds(v * _L, _L)]
        m = wv >= 0
        cnt = plsc.all_reduce_population_count(m)
        rank = plsc.cumsum(m.astype(jnp.int32)) - 1
        gdest = wid * _SLICE + v * _L + iota
        comb = gdest * _B + jnp.where(m, wv, 0)
        addr = jnp.where(m, base + rank, _SLICE + iota)
        plsc.store_scatter(cl_v, [addr], comb, mask=m)
        return base + cnt

    base = lax.fori_loop(0, _WCHUNK, compact_body,
                         jnp.zeros((_L,), jnp.int32), unroll=False)
    n = jnp.max(base)

    short_cp.wait()

    def emit_body(c, carry):
        @pl.when(c * _L < n)
        def _():
            cl = cl_v[pl.ds(c * _L, _L)]
            valid = (c * _L + iota) < n
            cm = jnp.max(jnp.where(valid, cl, -1))
            clf = jnp.where(valid, cl, cm)
            dest = lax.shift_right_logical(clf, _LOG_B)
            fp = clf & (_B - 1)
            g_cp = pltpu.make_async_copy(upd_hbm.at[fp], row_buf, sem_io)
            g_cp.start()
            g_cp.wait()
            s_cp = pltpu.make_async_copy(row_buf, out_hbm.at[dest], sem_io)
            s_cp.start()
            s_cp.wait()
        return carry

    lax.fori_loop(0, _WCHUNK, emit_body, 0, unroll=False)

    med_cp.wait()
    long_cp.wait()


@functools.partial(jax.jit)
def _sc_kernel(updates, idx32, short_mem, medium_mem, long_mem):
    mesh = plsc.VectorSubcoreMesh(core_axis_name="c", subcore_axis_name="s")
    k = pl.kernel(
        _sc_body,
        out_type=jax.ShapeDtypeStruct((_TOTAL, _DIM), jnp.float32),
        mesh=mesh,
        scratch_types=[
            pltpu.VMEM((_B,), jnp.int32),
            pltpu.VMEM((_SLICE + _L,), jnp.int32),
            pltpu.VMEM((_SLICE + _L,), jnp.int32),
            pltpu.VMEM((_L,), jnp.int32),
            pltpu.VMEM((_L, _DIM), jnp.float32),
            pltpu.SemaphoreType.DMA,
            pltpu.SemaphoreType.DMA,
            pltpu.SemaphoreType.DMA,
        ],
        compiler_params=pltpu.CompilerParams(needs_layout_passes=False),
    )
    return k(updates, idx32, short_mem, medium_mem, long_mem)


def kernel(updates, short_idx, short_mem, medium_mem, long_mem):
    return _sc_kernel(updates, short_idx.astype(jnp.int32),
                      short_mem, medium_mem, long_mem)


# speedup vs baseline: 22.3844x; 22.3844x over previous
import functools

import jax
import jax.numpy as jnp
from jax import lax
from jax.experimental import pallas as pl
from jax.experimental.pallas import tpu as pltpu
from jax.experimental.pallas import tpu_sc as plsc

_SHORT = 65536
_MED = 16384
_LONG = 4096
_DIM = 512
_TOTAL = _SHORT + _MED + _LONG
_B = 8192

_L = 16
_NC = 2
_NS = 16
_NW = _NC * _NS

_SLICE = _SHORT // _NW
_NCHUNK = _B // _L
_WCHUNK = _SLICE // _L

_LOG_B = 13
_LOG_SLICE = 11

_BLK = 512
_N_SHORT = _SHORT // _BLK
_N_MED = _MED // _BLK
_N_LONG = _LONG // _BLK
_N_TOT = _TOTAL // _BLK


def _assemble_body(short_ref, med_ref, long_ref, out_ref):
    i = pl.program_id(0)

    @pl.when(i < _N_SHORT)
    def _():
        out_ref[...] = short_ref[...]

    @pl.when(jnp.logical_and(i >= _N_SHORT, i < _N_SHORT + _N_MED))
    def _():
        out_ref[...] = med_ref[...]

    @pl.when(i >= _N_SHORT + _N_MED)
    def _():
        out_ref[...] = long_ref[...]


def _assemble(short_mem, medium_mem, long_mem):
    return pl.pallas_call(
        _assemble_body,
        grid=(_N_TOT,),
        in_specs=[
            pl.BlockSpec((_BLK, _DIM), lambda i: (jnp.minimum(i, _N_SHORT - 1), 0)),
            pl.BlockSpec(
                (_BLK, _DIM),
                lambda i: (jnp.clip(i - _N_SHORT, 0, _N_MED - 1), 0),
            ),
            pl.BlockSpec(
                (_BLK, _DIM),
                lambda i: (jnp.clip(i - _N_SHORT - _N_MED, 0, _N_LONG - 1), 0),
            ),
        ],
        out_specs=pl.BlockSpec((_BLK, _DIM), lambda i: (i, 0)),
        out_shape=jax.ShapeDtypeStruct((_TOTAL, _DIM), jnp.float32),
    )(short_mem, medium_mem, long_mem)


def _sc_body(upd_hbm, idx_hbm, out_hbm,
             idx_v, w_v, cl_v, shift_v, row_buf, sem_idx, sem_io):
    cid = lax.axis_index("c")
    sid = lax.axis_index("s")
    wid = sid * _NC + cid

    idx_cp = pltpu.make_async_copy(idx_hbm, idx_v, sem_idx)
    idx_cp.start()
    idx_cp.wait()

    iota = lax.iota(jnp.int32, _L)
    minus1 = jnp.full((_L,), -1, jnp.int32)

    def init_body(v, carry):
        w_v[pl.ds(v * _L, _L)] = minus1
        return carry

    lax.fori_loop(0, _WCHUNK, init_body, 0, unroll=False)

    def scan_body(c, carry):
        iv = idx_v[pl.ds(c * _L, _L)]
        shift_v[...] = iv
        drop = iota < 0
        for sh in range(1, _L):
            nb = plsc.load_gather(shift_v, [jnp.minimum(iota + sh, _L - 1)])
            drop = drop | ((iv == nb) & (iota + sh < _L))
        mine = lax.shift_right_logical(iv, _LOG_SLICE) == wid
        keep = mine & jnp.logical_not(drop)
        addr = jnp.where(keep, iv & (_SLICE - 1), _SLICE + iota)
        plsc.store_scatter(w_v, [addr], iota + c * _L)
        return carry

    lax.fori_loop(0, _NCHUNK, scan_body, 0, unroll=False)

    def compact_body(v, base):
        wv = w_v[pl.ds(v * _L, _L)]
        m = wv >= 0
        cnt = plsc.all_reduce_population_count(m)
        rank = plsc.cumsum(m.astype(jnp.int32)) - 1
        gdest = wid * _SLICE + v * _L + iota
        comb = gdest * _B + jnp.where(m, wv, 0)
        addr = jnp.where(m, base + rank, _SLICE + iota)
        plsc.store_scatter(cl_v, [addr], comb, mask=m)
        return base + cnt

    base = lax.fori_loop(0, _WCHUNK, compact_body,
                         jnp.zeros((_L,), jnp.int32), unroll=False)
    n = jnp.max(base)

    def emit_body(c, carry):
        @pl.when(c * _L < n)
        def _():
            cl = cl_v[pl.ds(c * _L, _L)]
            valid = (c * _L + iota) < n
            cm = jnp.max(jnp.where(valid, cl, -1))
            clf = jnp.where(valid, cl, cm)
            dest = lax.shift_right_logical(clf, _LOG_B)
            fp = clf & (_B - 1)
            g_cp = pltpu.make_async_copy(upd_hbm.at[fp], row_buf, sem_io)
            g_cp.start()
            g_cp.wait()
            s_cp = pltpu.make_async_copy(row_buf, out_hbm.at[dest], sem_io)
            s_cp.start()
            s_cp.wait()
        return carry

    lax.fori_loop(0, _WCHUNK, emit_body, 0, unroll=False)


def _make_sc_scatter():
    mesh = plsc.VectorSubcoreMesh(core_axis_name="c", subcore_axis_name="s")
    return pl.kernel(
        _sc_body,
        out_type=(),
        mesh=mesh,
        scratch_types=[
            pltpu.VMEM((_B,), jnp.int32),
            pltpu.VMEM((_SLICE + _L,), jnp.int32),
            pltpu.VMEM((_SLICE + _L,), jnp.int32),
            pltpu.VMEM((_L,), jnp.int32),
            pltpu.VMEM((_L, _DIM), jnp.float32),
            pltpu.SemaphoreType.DMA,
            pltpu.SemaphoreType.DMA,
        ],
        compiler_params=pltpu.CompilerParams(needs_layout_passes=False),
    )


_sc_scatter = _make_sc_scatter()


def kernel(updates, short_idx, short_mem, medium_mem, long_mem):
    idx32 = short_idx.astype(jnp.int32)
    assembled = _assemble(short_mem, medium_mem, long_mem)
    out_ref = jax.new_ref(assembled)
    _sc_scatter(updates, idx32, out_ref)
    return jax.freeze(out_ref)
